# baseline (speedup 1.0000x reference)
import jax
import jax.numpy as jnp
from jax import lax
from jax.experimental import pallas as pl
from jax.experimental.pallas import tpu as pltpu
from jax.experimental.pallas import tpu_sc as plsc

N, C, H, W = 6, 96, 224, 224
HW = H * W
V = 100000
CONF = 32
OUT = 128

NC, NS = 2, 16
NW = NC * NS
CPW = C // NW

CHUNK = 6272
NCHUNK = HW // CHUNK
TOTCH = N * NCHUNK
UNROLL = 8
GROUPS = CHUNK // (16 * UNROLL)


def _sc_body(feats_hbm, ids_hbm, sums_hbm, cnts_hbm, acc, idbuf, valbuf, sems):
    wid = lax.axis_index("s") * NC + lax.axis_index("c")

    zeros16 = jnp.zeros((16,), jnp.float32)
    ones16 = jnp.ones((16,), jnp.float32)

    def zero_acc():
        def zb(i, carry):
            base = i * 128
            for u in range(8):
                acc[pl.ds(base + u * 16, 16)] = zeros16
            return carry
        lax.fori_loop(0, V // 128, zb, 0)
        tail = (V // 128) * 128
        for u in range((V - tail) // 16):
            acc[pl.ds(tail + u * 16, 16)] = zeros16

    def scatter_slot(b, use_vals):
        def gb(g, carry):
            base = g * (16 * UNROLL)
            for u in range(UNROLL):
                off = base + u * 16
                idx = idbuf[b, pl.ds(off, 16)]
                v = valbuf[b, pl.ds(off, 16)] if use_vals else ones16
                plsc.addupdate_scatter(acc, [idx], v)
            return carry
        lax.fori_loop(0, GROUPS, gb, 0)

    def start_ids(b, n, q):
        pltpu.async_copy(ids_hbm.at[pl.ds(n * HW + q * CHUNK, CHUNK)],
                         idbuf.at[b], sems.at[b])

    def start_vals(b, row, q):
        pltpu.async_copy(feats_hbm.at[pl.ds(row * HW + q * CHUNK, CHUNK)],
                         valbuf.at[b], sems.at[b + 2])

    def wait_ids(b):
        pltpu.make_async_copy(ids_hbm.at[pl.ds(0, CHUNK)],
                              idbuf.at[b], sems.at[b]).wait()

    def wait_vals(b):
        pltpu.make_async_copy(feats_hbm.at[pl.ds(0, CHUNK)],
                              valbuf.at[b], sems.at[b + 2]).wait()

    @pl.when(wid < N)
    def _():
        zero_acc()
        for b in range(2):
            start_ids(b, wid, b)

        def pair(i, carry):
            for b in range(2):
                t = 2 * i + b
                wait_ids(b)
                scatter_slot(b, False)
                nxt = t + 2

                @pl.when(nxt < NCHUNK)
                def _():
                    start_ids(b, wid, nxt)
            return carry
        lax.fori_loop(0, NCHUNK // 2, pair, 0)
        pltpu.sync_copy(acc, cnts_hbm.at[wid])

    for k in range(CPW):
        ch = wid * CPW + k
        zero_acc()
        for b in range(2):
            n0 = b // NCHUNK
            start_ids(b, n0, b % NCHUNK)
            start_vals(b, n0 * C + ch, b % NCHUNK)

        def pair(i, carry, ch=ch):
            for b in range(2):
                t = 2 * i + b
                wait_ids(b)
                wait_vals(b)
                scatter_slot(b, True)
                nxt = t + 2

                @pl.when(nxt < TOTCH)
                def _():
                    n = lax.shift_right_logical(nxt, 3)
                    q = lax.bitwise_and(nxt, NCHUNK - 1)
                    start_ids(b, n, q)
                    start_vals(b, n * C + ch, q)
            return carry
        lax.fori_loop(0, TOTCH // 2, pair, 0)
        pltpu.sync_copy(acc, sums_hbm.at[ch])


_sc_scatter = pl.kernel(
    _sc_body,
    out_type=[
        jax.ShapeDtypeStruct((C, V), jnp.float32),
        jax.ShapeDtypeStruct((N, V), jnp.float32),
    ],
    mesh=plsc.VectorSubcoreMesh(
        core_axis_name="c", subcore_axis_name="s",
        num_cores=NC, num_subcores=NS,
    ),
    scratch_types=[
        pltpu.VMEM((V,), jnp.float32),
        pltpu.VMEM((2, CHUNK), jnp.int32),
        pltpu.VMEM((2, CHUNK), jnp.float32),
        pltpu.SemaphoreType.DMA((4,)),
    ],
    compiler_params=pltpu.CompilerParams(
        use_tc_tiling_on_sc=True, needs_layout_passes=False),
)


VB = 2048


def _tc_body(sums_ref, cnts_ref, conf_ref, w_ref, b_ref, out_ref):
    s = sums_ref[...]
    cnt = jnp.sum(cnts_ref[...], axis=0, keepdims=True)
    sv = s / jnp.maximum(cnt, 1.0)
    w = w_ref[...]
    w1 = w[:, :C]
    w2 = w[:, C:]
    a = lax.dot_general(sv, w1, (((0,), (1,)), ((), ())),
                        preferred_element_type=jnp.float32)
    b2 = lax.dot_general(conf_ref[0], w2, (((1,), (1,)), ((), ())),
                         preferred_element_type=jnp.float32)
    out_ref[0] = a + b2 + b_ref[...]


_tc_mix = pl.pallas_call(
    _tc_body,
    grid=(pl.cdiv(V, VB),),
    in_specs=[
        pl.BlockSpec((C, VB), lambda i: (0, i)),
        pl.BlockSpec((N, VB), lambda i: (0, i)),
        pl.BlockSpec((1, VB, CONF), lambda i: (0, i, 0)),
        pl.BlockSpec((OUT, C + CONF), lambda i: (0, 0)),
        pl.BlockSpec((1, OUT), lambda i: (0, 0)),
    ],
    out_specs=pl.BlockSpec((1, VB, OUT), lambda i: (0, i, 0)),
    out_shape=jax.ShapeDtypeStruct((1, V, OUT), jnp.float32),
)


def kernel(camera_pose, padded_intrinsics, padded_img_features, depths,
           padding_confidence, out_voxel_ids, W_mix, b_mix):
    feats = padded_img_features.reshape(N * C * HW)
    ids = out_voxel_ids.reshape(N * HW).astype(jnp.int32)
    sums, cnts = _sc_scatter(feats, ids)
    return _tc_mix(sums, cnts, padding_confidence, W_mix, b_mix.reshape(1, OUT))


# speedup vs baseline: 1.0920x; 1.0920x over previous
import jax
import jax.numpy as jnp
from jax import lax
from jax.experimental import pallas as pl
from jax.experimental.pallas import tpu as pltpu
from jax.experimental.pallas import tpu_sc as plsc

N, C, H, W = 6, 96, 224, 224
HW = H * W
V = 100000
VR = 782
SLOTS = 104
CONF = 32
OUT = 128

NC, NS = 2, 16
NW = NC * NS
CPW = C // NW

CHUNK = 6272
NCHUNK = HW // CHUNK
TOTCH = N * NCHUNK
UNROLL = 8
GROUPS = CHUNK // (16 * UNROLL)


def _sc_body(feats_hbm, ids_hbm, sums_hbm, acc, idbuf, valbuf, sems):
    wid = lax.axis_index("s") * NC + lax.axis_index("c")

    zeros16 = jnp.zeros((16,), jnp.float32)
    ones16 = jnp.ones((16,), jnp.float32)
    zeros16i = jnp.zeros((16,), jnp.int32)

    def zero_acc():
        def zb(r, carry):
            for u in range(8):
                acc[r, 0, pl.ds(u * 16, 16)] = zeros16
            return carry
        lax.fori_loop(0, VR, zb, 0)

    def scatter_slot(b, use_vals):
        def gb(g, carry):
            base = g * (16 * UNROLL)
            for u in range(UNROLL):
                off = base + u * 16
                idx = idbuf[b, pl.ds(off, 16)]
                rows = lax.shift_right_logical(idx, 7)
                cols = lax.bitwise_and(idx, 127)
                v = valbuf[b, pl.ds(off, 16)] if use_vals else ones16
                plsc.addupdate_scatter(acc, [rows, zeros16i, cols], v)
            return carry
        lax.fori_loop(0, GROUPS, gb, 0)

    def start_ids(b, n, q):
        pltpu.async_copy(ids_hbm.at[pl.ds(n * HW + q * CHUNK, CHUNK)],
                         idbuf.at[b], sems.at[b])

    def start_vals(b, row, q):
        pltpu.async_copy(feats_hbm.at[pl.ds(row * HW + q * CHUNK, CHUNK)],
                         valbuf.at[b], sems.at[b + 2])

    def wait_ids(b):
        pltpu.make_async_copy(ids_hbm.at[pl.ds(0, CHUNK)],
                              idbuf.at[b], sems.at[b]).wait()

    def wait_vals(b):
        pltpu.make_async_copy(feats_hbm.at[pl.ds(0, CHUNK)],
                              valbuf.at[b], sems.at[b + 2]).wait()

    def write_slot(slot):
        pltpu.sync_copy(acc, sums_hbm.at[:, pl.ds(slot, 1), :])

    @pl.when(wid < N)
    def _():
        zero_acc()
        for b in range(2):
            start_ids(b, wid, b)

        def pair(i, carry):
            for b in range(2):
                t = 2 * i + b
                wait_ids(b)
                scatter_slot(b, False)
                nxt = t + 2

                @pl.when(nxt < NCHUNK)
                def _():
                    start_ids(b, wid, nxt)
            return carry
        lax.fori_loop(0, NCHUNK // 2, pair, 0)
        write_slot(C + wid)

    for k in range(CPW):
        ch = wid * CPW + k
        zero_acc()
        for b in range(2):
            start_ids(b, 0, b)
            start_vals(b, ch, b)

        def pair(i, carry, ch=ch):
            for b in range(2):
                t = 2 * i + b
                wait_ids(b)
                wait_vals(b)
                scatter_slot(b, True)
                nxt = t + 2

                @pl.when(nxt < TOTCH)
                def _():
                    n = lax.shift_right_logical(nxt, 3)
                    q = lax.bitwise_and(nxt, NCHUNK - 1)
                    start_ids(b, n, q)
                    start_vals(b, n * C + ch, q)
            return carry
        lax.fori_loop(0, TOTCH // 2, pair, 0)
        write_slot(ch)


_sc_scatter = pl.kernel(
    _sc_body,
    out_type=[
        jax.ShapeDtypeStruct((VR, SLOTS, 128), jnp.float32),
    ],
    mesh=plsc.VectorSubcoreMesh(
        core_axis_name="c", subcore_axis_name="s",
        num_cores=NC, num_subcores=NS,
    ),
    scratch_types=[
        pltpu.VMEM((VR, 1, 128), jnp.float32),
        pltpu.VMEM((2, CHUNK), jnp.int32),
        pltpu.VMEM((2, CHUNK), jnp.float32),
        pltpu.SemaphoreType.DMA((4,)),
    ],
    compiler_params=pltpu.CompilerParams(
        use_tc_tiling_on_sc=False, needs_layout_passes=False),
)


RB = 16
VB = RB * 128


def _tc_body(sums_ref, conf_ref, w_ref, b_ref, out_ref):
    w = w_ref[...]
    w1 = w[:, :C]
    w2 = w[:, C:]
    bias = b_ref[...]
    for t in range(RB):
        blk = sums_ref[t]
        cnt = jnp.sum(blk[C:C + N], axis=0, keepdims=True)
        sv = blk[:C] / jnp.maximum(cnt, 1.0)
        a = lax.dot_general(sv, w1, (((0,), (1,)), ((), ())),
                            preferred_element_type=jnp.float32)
        cf = conf_ref[0, pl.ds(t * 128, 128), :]
        b2 = lax.dot_general(cf, w2, (((1,), (1,)), ((), ())),
                             preferred_element_type=jnp.float32)
        out_ref[0, pl.ds(t * 128, 128), :] = a + b2 + bias


_tc_mix = pl.pallas_call(
    _tc_body,
    grid=(pl.cdiv(VR, RB),),
    in_specs=[
        pl.BlockSpec((RB, SLOTS, 128), lambda i: (i, 0, 0)),
        pl.BlockSpec((1, VB, CONF), lambda i: (0, i, 0)),
        pl.BlockSpec((OUT, C + CONF), lambda i: (0, 0)),
        pl.BlockSpec((1, OUT), lambda i: (0, 0)),
    ],
    out_specs=pl.BlockSpec((1, VB, OUT), lambda i: (0, i, 0)),
    out_shape=jax.ShapeDtypeStruct((1, V, OUT), jnp.float32),
)


def kernel(camera_pose, padded_intrinsics, padded_img_features, depths,
           padding_confidence, out_voxel_ids, W_mix, b_mix):
    feats = padded_img_features.reshape(N * C * HW)
    ids = out_voxel_ids.reshape(N * HW).astype(jnp.int32)
    (sums,) = _sc_scatter(feats, ids)
    return _tc_mix(sums, padding_confidence, W_mix, b_mix.reshape(1, OUT))


# speedup vs baseline: 1.1105x; 1.0169x over previous
import jax
import jax.numpy as jnp
from jax import lax
from jax.experimental import pallas as pl
from jax.experimental.pallas import tpu as pltpu
from jax.experimental.pallas import tpu_sc as plsc

N, C, H, W = 6, 96, 224, 224
HW = H * W
V = 100000
VR = 782
SLOTS = 128
CONF = 32
OUT = 128

NC, NS = 2, 16
NW = NC * NS
CPW = C // NW

CHUNK = 6272
NCHUNK = HW // CHUNK
TOTCH = N * NCHUNK
UNROLL = 8
GROUPS = CHUNK // (16 * UNROLL)


def _sc_body(feats_hbm, ids_hbm, sums_hbm, acc, idbuf, valbuf, sems):
    wid = lax.axis_index("s") * NC + lax.axis_index("c")

    zeros16 = jnp.zeros((16,), jnp.float32)
    ones16 = jnp.ones((16,), jnp.float32)
    zeros16i = jnp.zeros((16,), jnp.int32)

    def zero_acc():
        def zb(r, carry):
            for u in range(8):
                acc[r, 0, pl.ds(u * 16, 16)] = zeros16
            return carry
        lax.fori_loop(0, VR, zb, 0)

    def scatter_slot(b, use_vals):
        def gb(g, carry):
            base = g * (16 * UNROLL)
            for u in range(UNROLL):
                off = base + u * 16
                idx = idbuf[b, pl.ds(off, 16)]
                rows = lax.shift_right_logical(idx, 7)
                cols = lax.bitwise_and(idx, 127)
                v = valbuf[b, pl.ds(off, 16)] if use_vals else ones16
                plsc.addupdate_scatter(acc, [rows, zeros16i, cols], v)
            return carry
        lax.fori_loop(0, GROUPS, gb, 0)

    def start_ids(b, n, q):
        pltpu.async_copy(ids_hbm.at[pl.ds(n * HW + q * CHUNK, CHUNK)],
                         idbuf.at[b], sems.at[b])

    def start_vals(b, row, q):
        pltpu.async_copy(feats_hbm.at[pl.ds(row * HW + q * CHUNK, CHUNK)],
                         valbuf.at[b], sems.at[b + 2])

    def wait_ids(b):
        pltpu.make_async_copy(ids_hbm.at[pl.ds(0, CHUNK)],
                              idbuf.at[b], sems.at[b]).wait()

    def wait_vals(b):
        pltpu.make_async_copy(feats_hbm.at[pl.ds(0, CHUNK)],
                              valbuf.at[b], sems.at[b + 2]).wait()

    def write_slot(slot):
        pltpu.sync_copy(acc, sums_hbm.at[:, pl.ds(slot, 1), :])

    CNT_PIX = N * HW // NW
    CNT_SUB = CNT_PIX // 2
    zero_acc()
    for b in range(2):
        pltpu.async_copy(
            ids_hbm.at[pl.ds(wid * CNT_PIX + b * CNT_SUB, CNT_SUB)],
            idbuf.at[b, pl.ds(0, CNT_SUB)], sems.at[b])
    for b in range(2):
        pltpu.make_async_copy(ids_hbm.at[pl.ds(0, CNT_SUB)],
                              idbuf.at[b, pl.ds(0, CNT_SUB)],
                              sems.at[b]).wait()

        def cgb(g, carry, b=b):
            base = g * (16 * 7)
            for u in range(7):
                off = base + u * 16
                idx = idbuf[b, pl.ds(off, 16)]
                rows = lax.shift_right_logical(idx, 7)
                cols = lax.bitwise_and(idx, 127)
                plsc.addupdate_scatter(acc, [rows, zeros16i, cols], ones16)
            return carry
        lax.fori_loop(0, CNT_SUB // (16 * 7), cgb, 0)
    write_slot(C + wid)

    for k in range(CPW):
        ch = wid * CPW + k
        zero_acc()
        for b in range(2):
            start_ids(b, 0, b)
            start_vals(b, ch, b)

        def pair(i, carry, ch=ch):
            for b in range(2):
                t = 2 * i + b
                wait_ids(b)
                wait_vals(b)
                scatter_slot(b, True)
                nxt = t + 2

                @pl.when(nxt < TOTCH)
                def _():
                    n = lax.shift_right_logical(nxt, 3)
                    q = lax.bitwise_and(nxt, NCHUNK - 1)
                    start_ids(b, n, q)
                    start_vals(b, n * C + ch, q)
            return carry
        lax.fori_loop(0, TOTCH // 2, pair, 0)
        write_slot(ch)


_sc_scatter = pl.kernel(
    _sc_body,
    out_type=[
        jax.ShapeDtypeStruct((VR, SLOTS, 128), jnp.float32),
    ],
    mesh=plsc.VectorSubcoreMesh(
        core_axis_name="c", subcore_axis_name="s",
        num_cores=NC, num_subcores=NS,
    ),
    scratch_types=[
        pltpu.VMEM((VR, 1, 128), jnp.float32),
        pltpu.VMEM((2, CHUNK), jnp.int32),
        pltpu.VMEM((2, CHUNK), jnp.float32),
        pltpu.SemaphoreType.DMA((4,)),
    ],
    compiler_params=pltpu.CompilerParams(
        use_tc_tiling_on_sc=False, needs_layout_passes=False),
)


RB = 16
VB = RB * 128


def _tc_body(sums_ref, conf_ref, w_ref, b_ref, out_ref):
    w = w_ref[...]
    w1 = w[:, :C]
    w2 = w[:, C:]
    bias = b_ref[...]
    for t in range(RB):
        blk = sums_ref[t]
        cnt = jnp.sum(blk[C:SLOTS], axis=0, keepdims=True)
        sv = blk[:C] / jnp.maximum(cnt, 1.0)
        a = lax.dot_general(sv, w1, (((0,), (1,)), ((), ())),
                            preferred_element_type=jnp.float32)
        cf = conf_ref[0, pl.ds(t * 128, 128), :]
        b2 = lax.dot_general(cf, w2, (((1,), (1,)), ((), ())),
                             preferred_element_type=jnp.float32)
        out_ref[0, pl.ds(t * 128, 128), :] = a + b2 + bias


_tc_mix = pl.pallas_call(
    _tc_body,
    grid=(pl.cdiv(VR, RB),),
    in_specs=[
        pl.BlockSpec((RB, SLOTS, 128), lambda i: (i, 0, 0)),
        pl.BlockSpec((1, VB, CONF), lambda i: (0, i, 0)),
        pl.BlockSpec((OUT, C + CONF), lambda i: (0, 0)),
        pl.BlockSpec((1, OUT), lambda i: (0, 0)),
    ],
    out_specs=pl.BlockSpec((1, VB, OUT), lambda i: (0, i, 0)),
    out_shape=jax.ShapeDtypeStruct((1, V, OUT), jnp.float32),
)


def kernel(camera_pose, padded_intrinsics, padded_img_features, depths,
           padding_confidence, out_voxel_ids, W_mix, b_mix):
    feats = padded_img_features.reshape(N * C * HW)
    ids = out_voxel_ids.reshape(N * HW).astype(jnp.int32)
    (sums,) = _sc_scatter(feats, ids)
    return _tc_mix(sums, padding_confidence, W_mix, b_mix.reshape(1, OUT))


# speedup vs baseline: 1.1120x; 1.0014x over previous
import jax
import jax.numpy as jnp
from jax import lax
from jax.experimental import pallas as pl
from jax.experimental.pallas import tpu as pltpu
from jax.experimental.pallas import tpu_sc as plsc

N, C, H, W = 6, 96, 224, 224
HW = H * W
V = 100000
VR = 782
SLOTS = 128
CONF = 32
OUT = 128

NC, NS = 2, 16
NW = NC * NS
CPW = C // NW

CHUNK = 6272
CROWS = CHUNK // 128
RPP = HW // 128
NCHUNK = HW // CHUNK
TOTCH = N * NCHUNK
UNROLL = 8


def _sc_body(feats_hbm, ids_hbm, sums_hbm, acc, idbuf, valbuf, sems):
    wid = lax.axis_index("s") * NC + lax.axis_index("c")

    zeros16 = jnp.zeros((16,), jnp.float32)
    ones16 = jnp.ones((16,), jnp.float32)
    zeros16i = jnp.zeros((16,), jnp.int32)

    def zero_acc():
        def zb(r, carry):
            for u in range(8):
                acc[r, 0, pl.ds(u * 16, 16)] = zeros16
            return carry
        lax.fori_loop(0, VR, zb, 0)

    def scatter_slot(b, use_vals):
        def gb(r, carry):
            for u in range(UNROLL):
                off = u * 16
                idx = idbuf[b, r, pl.ds(off, 16)]
                rows = lax.shift_right_logical(idx, 7)
                cols = lax.bitwise_and(idx, 127)
                v = valbuf[b, r, pl.ds(off, 16)] if use_vals else ones16
                plsc.addupdate_scatter(acc, [rows, zeros16i, cols], v)
            return carry
        lax.fori_loop(0, CROWS, gb, 0)

    def start_ids(b, n, q):
        pltpu.async_copy(
            ids_hbm.at[pl.ds(n * RPP + q * CROWS, CROWS), :],
            idbuf.at[b], sems.at[b])

    def start_vals(b, row, q):
        pltpu.async_copy(
            feats_hbm.at[pl.ds(row * RPP + q * CROWS, CROWS), :],
            valbuf.at[b], sems.at[b + 2])

    def wait_ids(b):
        pltpu.make_async_copy(ids_hbm.at[pl.ds(0, CROWS), :],
                              idbuf.at[b], sems.at[b]).wait()

    def wait_vals(b):
        pltpu.make_async_copy(feats_hbm.at[pl.ds(0, CROWS), :],
                              valbuf.at[b], sems.at[b + 2]).wait()

    def write_slot(slot):
        pltpu.sync_copy(acc, sums_hbm.at[:, pl.ds(slot, 1), :])

    zero_acc()
    for j in range((TOTCH + NW - 1) // NW):
        cix = wid + NW * j

        def do_count(cix=cix):
            pltpu.sync_copy(ids_hbm.at[pl.ds(cix * CROWS, CROWS), :],
                            idbuf.at[0])
            scatter_slot(0, False)

        if (j + 1) * NW > TOTCH:
            pl.when(cix < TOTCH)(do_count)
        else:
            do_count()
    write_slot(C + wid)

    for k in range(CPW):
        ch = wid * CPW + k
        zero_acc()
        for b in range(2):
            start_ids(b, 0, b)
            start_vals(b, ch, b)

        def pair(i, carry, ch=ch):
            for b in range(2):
                t = 2 * i + b
                wait_ids(b)
                wait_vals(b)
                scatter_slot(b, True)
                nxt = t + 2

                @pl.when(nxt < TOTCH)
                def _():
                    n = lax.shift_right_logical(nxt, 3)
                    q = lax.bitwise_and(nxt, NCHUNK - 1)
                    start_ids(b, n, q)
                    start_vals(b, n * C + ch, q)
            return carry
        lax.fori_loop(0, TOTCH // 2, pair, 0)
        write_slot(ch)


_sc_scatter = pl.kernel(
    _sc_body,
    out_type=[
        jax.ShapeDtypeStruct((VR, SLOTS, 128), jnp.float32),
    ],
    mesh=plsc.VectorSubcoreMesh(
        core_axis_name="c", subcore_axis_name="s",
        num_cores=NC, num_subcores=NS,
    ),
    scratch_types=[
        pltpu.VMEM((VR, 1, 128), jnp.float32),
        pltpu.VMEM((2, CROWS, 128), jnp.int32),
        pltpu.VMEM((2, CROWS, 128), jnp.float32),
        pltpu.SemaphoreType.DMA((4,)),
    ],
    compiler_params=pltpu.CompilerParams(
        use_tc_tiling_on_sc=False, needs_layout_passes=False),
)


RB = 16
VB = RB * 128


def _tc_body(sums_ref, conf_ref, w_ref, b_ref, out_ref):
    w = w_ref[...]
    w1 = w[:, :C]
    w2 = w[:, C:]
    bias = b_ref[...]
    for t in range(RB):
        blk = sums_ref[t]
        cnt = jnp.sum(blk[C:SLOTS], axis=0, keepdims=True)
        sv = blk[:C] / jnp.maximum(cnt, 1.0)
        a = lax.dot_general(sv, w1, (((0,), (1,)), ((), ())),
                            preferred_element_type=jnp.float32)
        cf = conf_ref[0, pl.ds(t * 128, 128), :]
        b2 = lax.dot_general(cf, w2, (((1,), (1,)), ((), ())),
                             preferred_element_type=jnp.float32)
        out_ref[0, pl.ds(t * 128, 128), :] = a + b2 + bias


_tc_mix = pl.pallas_call(
    _tc_body,
    grid=(pl.cdiv(VR, RB),),
    in_specs=[
        pl.BlockSpec((RB, SLOTS, 128), lambda i: (i, 0, 0)),
        pl.BlockSpec((1, VB, CONF), lambda i: (0, i, 0)),
        pl.BlockSpec((OUT, C + CONF), lambda i: (0, 0)),
        pl.BlockSpec((1, OUT), lambda i: (0, 0)),
    ],
    out_specs=pl.BlockSpec((1, VB, OUT), lambda i: (0, i, 0)),
    out_shape=jax.ShapeDtypeStruct((1, V, OUT), jnp.float32),
)


def kernel(camera_pose, padded_intrinsics, padded_img_features, depths,
           padding_confidence, out_voxel_ids, W_mix, b_mix):
    feats = padded_img_features.reshape(N * C * HW // 128, 128)
    ids = out_voxel_ids.reshape(N * HW // 128, 128).astype(jnp.int32)
    (sums,) = _sc_scatter(feats, ids)
    return _tc_mix(sums, padding_confidence, W_mix, b_mix.reshape(1, OUT))
